# Initial kernel scaffold; baseline (speedup 1.0000x reference)
import jax
import jax.numpy as jnp
from jax.experimental import pallas as pl


def kernel(node_embeddings, edge_index, lin_w, lin_b, curvature):
    raise NotImplementedError("write your pallas kernel here")


# speedup vs baseline: 4.0986x; 4.0986x over previous
import functools

import jax
import jax.numpy as jnp
from jax import lax
from jax.experimental import pallas as pl
from jax.experimental.pallas import tpu as pltpu
from jax.experimental.pallas import tpu_sc as plsc

N = 10000
NP = 10240
E = 160000
D = 256
DH = D // 2
NSC = 16
EPS = E // NSC
K = 80
NB = EPS // K
RPS = NP // NSC
ZR = 128
RB = 1000
RB3 = 1024


def _stage1_body(x_ref, w_ref, b_ref, c_ref, t0_ref, t1_ref):
    x = x_ref[...]
    c = c_ref[0, 0]
    sq = jnp.sqrt(c)
    nrm = jnp.sqrt(jnp.sum(x * x, axis=1, keepdims=True))
    z = sq * nrm
    atz = 0.5 * jnp.log((1.0 + z) / (1.0 - z))
    tang = x * (2.0 / sq * atz / nrm)
    res = lax.dot_general(tang, w_ref[...], (((1,), (1,)), ((), ())),
                          preferred_element_type=jnp.float32) + b_ref[...]
    t0_ref[...] = res[:, :DH]
    t1_ref[...] = res[:, DH:]


def _transform(x, w, b2, c2):
    return pl.pallas_call(
        _stage1_body,
        grid=(N // RB,),
        in_specs=[
            pl.BlockSpec((RB, D), lambda i: (i, 0)),
            pl.BlockSpec((D, D), lambda i: (0, 0)),
            pl.BlockSpec((1, D), lambda i: (0, 0)),
            pl.BlockSpec(memory_space=pltpu.SMEM),
        ],
        out_specs=[
            pl.BlockSpec((RB, DH), lambda i: (i, 0)),
            pl.BlockSpec((RB, DH), lambda i: (i, 0)),
        ],
        out_shape=[
            jax.ShapeDtypeStruct((N, DH), jnp.float32),
            jax.ShapeDtypeStruct((N, DH), jnp.float32),
        ],
    )(x, w, b2, c2)


def _sc_body(t0_hbm, t1_hbm, src_hbm, dst_hbm, s0_hbm, s1_hbm, cnt_hbm,
             acc, cacc, sidx, didx, rows, ones, zb, zcb, sem):
    cid = lax.axis_index("c")
    sid = lax.axis_index("s")
    base = sid * RPS

    def fz(i, _):
        for j in range(8):
            zb[i, pl.ds(j * 16, 16)] = jnp.zeros((16,), jnp.float32)
        return 0
    lax.fori_loop(0, ZR, fz, 0)

    def fzc(i, _):
        zcb[i] = jnp.zeros((16,), jnp.float32)
        return 0
    lax.fori_loop(0, RPS, fzc, 0)

    def fo(i, _):
        ones[i] = jnp.ones((16,), jnp.float32)
        return 0
    lax.fori_loop(0, K, fo, 0)

    for k in range(RPS // ZR):
        pltpu.sync_copy(zb, acc.at[pl.ds(base + k * ZR, ZR)])

    @pl.when(cid == 0)
    def _():
        pltpu.sync_copy(zcb, cacc.at[pl.ds(base, RPS)])

    plsc.subcore_barrier()

    def do_edges(t_hbm, with_cnt):
        def body(i, _):
            pltpu.sync_copy(src_hbm.at[sid, i], sidx)
            pltpu.sync_copy(dst_hbm.at[sid, i], didx)
            pltpu.async_copy(t_hbm.at[sidx], rows, sem).wait()
            pltpu.sync_copy(rows, acc.at[didx], add=True)
            if with_cnt:
                pltpu.sync_copy(ones, cacc.at[didx], add=True)
            return 0
        lax.fori_loop(0, NB, body, 0)

    @pl.when(cid == 0)
    def _():
        do_edges(t0_hbm, True)
        plsc.subcore_barrier()
        pltpu.sync_copy(acc.at[pl.ds(base, RPS)], s0_hbm.at[pl.ds(base, RPS)])
        pltpu.sync_copy(cacc.at[pl.ds(base, RPS)], cnt_hbm.at[pl.ds(base, RPS)])

    @pl.when(cid == 1)
    def _():
        do_edges(t1_hbm, False)
        plsc.subcore_barrier()
        pltpu.sync_copy(acc.at[pl.ds(base, RPS)], s1_hbm.at[pl.ds(base, RPS)])


def _scatter_mean(t0, t1, src3, dst3):
    mesh = plsc.VectorSubcoreMesh(core_axis_name="c", subcore_axis_name="s")
    f = pl.kernel(
        _sc_body,
        out_type=[
            jax.ShapeDtypeStruct((NP, DH), jnp.float32),
            jax.ShapeDtypeStruct((NP, DH), jnp.float32),
            jax.ShapeDtypeStruct((NP, 16), jnp.float32),
        ],
        mesh=mesh,
        scratch_types=[
            pltpu.VMEM_SHARED((NP, DH), jnp.float32),
            pltpu.VMEM_SHARED((NP, 16), jnp.float32),
            pltpu.VMEM((K,), jnp.int32),
            pltpu.VMEM((K,), jnp.int32),
            pltpu.VMEM((K, DH), jnp.float32),
            pltpu.VMEM((K, 16), jnp.float32),
            pltpu.VMEM((ZR, DH), jnp.float32),
            pltpu.VMEM((RPS, 16), jnp.float32),
            pltpu.SemaphoreType.DMA,
        ],
        compiler_params=pltpu.CompilerParams(use_tc_tiling_on_sc=False),
    )
    return f(t0, t1, src3, dst3)


def _stage3_body(s0_ref, s1_ref, cnt_ref, c_ref, out_ref):
    s = jnp.concatenate([s0_ref[...], s1_ref[...]], axis=1)
    cntv = cnt_ref[:, 0:1]
    neigh = s / jnp.maximum(cntv, 1.0)
    c = c_ref[0, 0]
    sq = jnp.sqrt(c)
    nv = jnp.sqrt(jnp.sum(neigh * neigh, axis=1, keepdims=True))
    out_ref[...] = jnp.tanh(sq * nv * 0.5) * neigh / (sq * nv)


def _expmap(s0, s1, cnt, c2):
    return pl.pallas_call(
        _stage3_body,
        grid=(NP // RB3,),
        in_specs=[
            pl.BlockSpec((RB3, DH), lambda i: (i, 0)),
            pl.BlockSpec((RB3, DH), lambda i: (i, 0)),
            pl.BlockSpec((RB3, 16), lambda i: (i, 0)),
            pl.BlockSpec(memory_space=pltpu.SMEM),
        ],
        out_specs=pl.BlockSpec((RB3, D), lambda i: (i, 0)),
        out_shape=jax.ShapeDtypeStruct((NP, D), jnp.float32),
    )(s0, s1, cnt, c2)


def kernel(node_embeddings, edge_index, lin_w, lin_b, curvature):
    c2 = curvature.reshape(1, 1)
    b2 = lin_b.reshape(1, D)
    t0, t1 = _transform(node_embeddings, lin_w, b2, c2)
    src3 = edge_index[0].reshape(NSC, NB, K)
    dst3 = edge_index[1].reshape(NSC, NB, K)
    s0, s1, cnt = _scatter_mean(t0, t1, src3, dst3)
    return _expmap(s0, s1, cnt, c2)[:N]


# speedup vs baseline: 7.0550x; 1.7213x over previous
import functools

import jax
import jax.numpy as jnp
from jax import lax
from jax.experimental import pallas as pl
from jax.experimental.pallas import tpu as pltpu
from jax.experimental.pallas import tpu_sc as plsc

N = 10000
NP = 10240
E = 160000
D = 256
DH = D // 2
NSC = 16
EPS = E // NSC
K = 80
NB = EPS // K
NW = 25
RPS = NP // NSC
RB = 1000
RB3 = 1024


def _stage1_body(x_ref, w_ref, b_ref, c_ref, t0_ref, t1_ref):
    x = x_ref[...]
    c = c_ref[0, 0]
    sq = jnp.sqrt(c)
    nrm = jnp.sqrt(jnp.sum(x * x, axis=1, keepdims=True))
    z = sq * nrm
    atz = 0.5 * jnp.log((1.0 + z) / (1.0 - z))
    tang = x * (2.0 / sq * atz / nrm)
    res = lax.dot_general(tang, w_ref[...], (((1,), (1,)), ((), ())),
                          preferred_element_type=jnp.float32) + b_ref[...]
    t0_ref[...] = res[:, :DH]
    t1_ref[...] = res[:, DH:]


def _transform(x, w, b2, c2):
    return pl.pallas_call(
        _stage1_body,
        grid=(N // RB,),
        in_specs=[
            pl.BlockSpec((RB, D), lambda i: (i, 0)),
            pl.BlockSpec((D, D), lambda i: (0, 0)),
            pl.BlockSpec((1, D), lambda i: (0, 0)),
            pl.BlockSpec(memory_space=pltpu.SMEM),
        ],
        out_specs=[
            pl.BlockSpec((RB, DH), lambda i: (i, 0)),
            pl.BlockSpec((RB, DH), lambda i: (i, 0)),
        ],
        out_shape=[
            jax.ShapeDtypeStruct((N, DH), jnp.float32),
            jax.ShapeDtypeStruct((N, DH), jnp.float32),
        ],
    )(x, w, b2, c2)


def _sc_body(t0_hbm, t1_hbm, src_hbm, dst_hbm, s0_hbm, s1_hbm, cnt_hbm,
             acc, cacc, sidx, didx, rows_a, rows_b, ones, zcb,
             sem_a, sem_b):
    cid = lax.axis_index("c")
    sid = lax.axis_index("s")
    base = sid * RPS

    def fz(i, _):
        for j in range(DH // 16):
            rows_a[i, pl.ds(j * 16, 16)] = jnp.zeros((16,), jnp.float32)
        zcb[i] = jnp.zeros((16,), jnp.float32)
        ones[i] = jnp.ones((16,), jnp.float32)
        return 0
    lax.fori_loop(0, K, fz, 0)

    for k in range(RPS // K):
        pltpu.sync_copy(rows_a, acc.at[pl.ds(base + k * K, K)])

    @pl.when(cid == 0)
    def _():
        for k in range(RPS // K):
            pltpu.sync_copy(zcb, cacc.at[pl.ds(base + k * K, K)])

    plsc.subcore_barrier()

    def do_edges(t_hbm, with_cnt):
        def gather(i, buf, sem):
            pltpu.async_copy(t_hbm.at[sidx.at[i]], buf, sem)

        def gwait(buf, sem):
            pltpu.make_async_copy(t_hbm.at[pl.ds(0, K)], buf, sem).wait()

        def scatter(i, buf):
            pltpu.sync_copy(buf, acc.at[didx.at[i]], add=True)
            if with_cnt:
                pltpu.sync_copy(ones, cacc.at[didx.at[i]], add=True)

        def window(w, _):
            pltpu.sync_copy(src_hbm.at[sid, pl.ds(w * NW, NW)], sidx)
            pltpu.sync_copy(dst_hbm.at[sid, pl.ds(w * NW, NW)], didx)
            gather(0, rows_a, sem_a)

            def body(j, _):
                i0 = 2 * j
                gwait(rows_a, sem_a)
                gather(i0 + 1, rows_b, sem_b)
                scatter(i0, rows_a)
                gwait(rows_b, sem_b)
                gather(i0 + 2, rows_a, sem_a)
                scatter(i0 + 1, rows_b)
                return 0
            lax.fori_loop(0, (NW - 1) // 2, body, 0)
            gwait(rows_a, sem_a)
            scatter(NW - 1, rows_a)
            return 0
        lax.fori_loop(0, NB // NW, window, 0)

    @pl.when(cid == 0)
    def _():
        do_edges(t0_hbm, True)
        plsc.subcore_barrier()
        pltpu.sync_copy(acc.at[pl.ds(base, RPS)], s0_hbm.at[pl.ds(base, RPS)])
        pltpu.sync_copy(cacc.at[pl.ds(base, RPS)], cnt_hbm.at[pl.ds(base, RPS)])

    @pl.when(cid == 1)
    def _():
        do_edges(t1_hbm, False)
        plsc.subcore_barrier()
        pltpu.sync_copy(acc.at[pl.ds(base, RPS)], s1_hbm.at[pl.ds(base, RPS)])


def _scatter_mean(t0, t1, src3, dst3):
    mesh = plsc.VectorSubcoreMesh(core_axis_name="c", subcore_axis_name="s")
    f = pl.kernel(
        _sc_body,
        out_type=[
            jax.ShapeDtypeStruct((NP, DH), jnp.float32),
            jax.ShapeDtypeStruct((NP, DH), jnp.float32),
            jax.ShapeDtypeStruct((NP, 16), jnp.float32),
        ],
        mesh=mesh,
        scratch_types=[
            pltpu.VMEM_SHARED((NP, DH), jnp.float32),
            pltpu.VMEM_SHARED((NP, 16), jnp.float32),
            pltpu.VMEM((NW, K), jnp.int32),
            pltpu.VMEM((NW, K), jnp.int32),
            pltpu.VMEM((K, DH), jnp.float32),
            pltpu.VMEM((K, DH), jnp.float32),
            pltpu.VMEM((K, 16), jnp.float32),
            pltpu.VMEM((K, 16), jnp.float32),
            pltpu.SemaphoreType.DMA,
            pltpu.SemaphoreType.DMA,
        ],
        compiler_params=pltpu.CompilerParams(use_tc_tiling_on_sc=False),
    )
    return f(t0, t1, src3, dst3)


def _stage3_body(s0_ref, s1_ref, cnt_ref, c_ref, out_ref):
    s = jnp.concatenate([s0_ref[...], s1_ref[...]], axis=1)
    cntv = cnt_ref[:, 0:1]
    neigh = s / jnp.maximum(cntv, 1.0)
    c = c_ref[0, 0]
    sq = jnp.sqrt(c)
    nv = jnp.sqrt(jnp.sum(neigh * neigh, axis=1, keepdims=True))
    out_ref[...] = jnp.tanh(sq * nv * 0.5) * neigh / (sq * nv)


def _expmap(s0, s1, cnt, c2):
    return pl.pallas_call(
        _stage3_body,
        grid=(NP // RB3,),
        in_specs=[
            pl.BlockSpec((RB3, DH), lambda i: (i, 0)),
            pl.BlockSpec((RB3, DH), lambda i: (i, 0)),
            pl.BlockSpec((RB3, 16), lambda i: (i, 0)),
            pl.BlockSpec(memory_space=pltpu.SMEM),
        ],
        out_specs=pl.BlockSpec((RB3, D), lambda i: (i, 0)),
        out_shape=jax.ShapeDtypeStruct((NP, D), jnp.float32),
    )(s0, s1, cnt, c2)


def kernel(node_embeddings, edge_index, lin_w, lin_b, curvature):
    c2 = curvature.reshape(1, 1)
    b2 = lin_b.reshape(1, D)
    t0, t1 = _transform(node_embeddings, lin_w, b2, c2)
    src3 = edge_index[0].reshape(NSC, NB, K)
    dst3 = edge_index[1].reshape(NSC, NB, K)
    s0, s1, cnt = _scatter_mean(t0, t1, src3, dst3)
    return _expmap(s0, s1, cnt, c2)[:N]
